# baseline (speedup 1.0000x reference)
import functools

import jax
import jax.numpy as jnp
from jax import lax
from jax.experimental import pallas as pl
from jax.experimental.pallas import tpu as pltpu
from jax.experimental.pallas import tpu_sc as plsc

NC, NS = 2, 16
NW = NC * NS

B, T = 16384, 200
V, D = 1000, 50
DP = 128
N = B * T
B_PER_W = N // NW
CHUNK = 128
NB = 4
NCHUNK = B_PER_W // CHUNK
NOUTER = NCHUNK // NB
L = 16
NGRP = CHUNK * D // L
UNROLL = 8

_mesh = plsc.VectorSubcoreMesh(core_axis_name="c", subcore_axis_name="s")


@functools.partial(
    pl.kernel,
    out_type=jax.ShapeDtypeStruct((N * D,), jnp.float32),
    mesh=_mesh,
    compiler_params=pltpu.CompilerParams(needs_layout_passes=False),
    scratch_types=[
        pltpu.VMEM((NB * CHUNK,), jnp.int32),
        pltpu.VMEM((NB, CHUNK, DP), jnp.float32),
        pltpu.VMEM((NB * CHUNK * D,), jnp.float32),
        pltpu.VMEM((CHUNK * D,), jnp.int32),
        pltpu.SemaphoreType.DMA,
        pltpu.SemaphoreType.DMA,
    ],
)
def _gather_kernel(idx_hbm, table_hbm, pmap_hbm, out_hbm, idx_v, rows_v, pack_v,
                   pmap_v, gsem, wsem):
    wid = lax.axis_index("s") * NC + lax.axis_index("c")
    base = wid * B_PER_W

    pltpu.sync_copy(pmap_hbm, pmap_v)

    def body(g, carry):
        pltpu.sync_copy(idx_hbm.at[pl.ds(base + g * NB * CHUNK, NB * CHUNK)], idx_v)
        gathers = [
            pltpu.async_copy(
                table_hbm.at[idx_v.at[pl.ds(b * CHUNK, CHUNK)]], rows_v.at[b], gsem
            )
            for b in range(NB)
        ]
        writes = []
        for b in range(NB):
            gathers[b].wait()

            iota = lax.iota(jnp.int32, L)
            tail_mask = iota < (D - 48)

            def pack(i, c):
                for u in range(UNROLL):
                    r = i * UNROLL + u
                    pbase = b * (CHUNK * D) + r * D
                    for s in (0, 16, 32, 48):
                        x = rows_v[b, r, pl.ds(s, L)]
                        addr = iota + (pbase + s)
                        if s == 48:
                            plsc.store_scatter(pack_v, [addr], x,
                                               mask=tail_mask)
                        else:
                            plsc.store_scatter(pack_v, [addr], x)
                return c

            lax.fori_loop(0, CHUNK // UNROLL, pack, 0)

            off = (base + (g * NB + b) * CHUNK) * D
            writes.append(
                pltpu.async_copy(
                    pack_v.at[pl.ds(b * CHUNK * D, CHUNK * D)],
                    out_hbm.at[pl.ds(off, CHUNK * D)],
                    wsem,
                )
            )
        for w in writes:
            w.wait()
        return carry

    lax.fori_loop(0, NOUTER, body, 0)


def kernel(pos_ids, table):
    idx = pos_ids.reshape(N).astype(jnp.int32)
    table_p = jnp.pad(table, ((0, 0), (0, DP - D)))
    p = jnp.arange(CHUNK * D, dtype=jnp.int32)
    pmap = (p // D) * DP + p % D
    out = _gather_kernel(idx, table_p, pmap)
    return out.reshape(B, T, D)


# speedup vs baseline: 1.3889x; 1.3889x over previous
import functools

import jax
import jax.numpy as jnp
from jax import lax
from jax.experimental import pallas as pl
from jax.experimental.pallas import tpu as pltpu
from jax.experimental.pallas import tpu_sc as plsc

NC, NS = 2, 16
NW = NC * NS

B, T = 16384, 200
V, D = 1000, 50
N = B * T
B_PER_W = N // NW
CHUNK = 128
NB = 4
NCHUNK = B_PER_W // CHUNK
NOUTER = NCHUNK // NB

_mesh = plsc.VectorSubcoreMesh(core_axis_name="c", subcore_axis_name="s")


@functools.partial(
    pl.kernel,
    out_type=jax.ShapeDtypeStruct((N, D), jnp.float32),
    mesh=_mesh,
    compiler_params=pltpu.CompilerParams(
        use_tc_tiling_on_sc=False,
        needs_layout_passes=False,
    ),
    scratch_types=[
        pltpu.VMEM((NB * CHUNK,), jnp.int32),
        pltpu.VMEM((NB, CHUNK, D), jnp.float32),
        pltpu.SemaphoreType.DMA,
        pltpu.SemaphoreType.DMA,
    ],
)
def _gather_kernel(idx_hbm, table_hbm, out_hbm, idx_v, rows_v, gsem, wsem):
    wid = lax.axis_index("s") * NC + lax.axis_index("c")
    base = wid * B_PER_W

    def body(g, carry):
        pltpu.sync_copy(idx_hbm.at[pl.ds(base + g * NB * CHUNK, NB * CHUNK)], idx_v)
        gathers = [
            pltpu.async_copy(
                table_hbm.at[idx_v.at[pl.ds(b * CHUNK, CHUNK)]], rows_v.at[b], gsem
            )
            for b in range(NB)
        ]
        writes = []
        for b in range(NB):
            gathers[b].wait()
            off = base + (g * NB + b) * CHUNK
            writes.append(
                pltpu.async_copy(rows_v.at[b], out_hbm.at[pl.ds(off, CHUNK)], wsem)
            )
        for w in writes:
            w.wait()
        return carry

    lax.fori_loop(0, NOUTER, body, 0)


def kernel(pos_ids, table):
    idx = pos_ids.reshape(N).astype(jnp.int32)
    out = _gather_kernel(idx, table)
    return out.reshape(B, T, D)


# speedup vs baseline: 3.1431x; 2.2630x over previous
import functools

import jax
import jax.numpy as jnp
from jax import lax
from jax.experimental import pallas as pl
from jax.experimental.pallas import tpu as pltpu
from jax.experimental.pallas import tpu_sc as plsc

NC, NS = 2, 16
NW = NC * NS

B, T = 16384, 200
V, D = 1000, 50
DP = 128
N = B * T
B_PER_W = N // NW
CHUNK = 128
NB = 4
NCHUNK = B_PER_W // CHUNK
NOUTER = NCHUNK // NB

_mesh = plsc.VectorSubcoreMesh(core_axis_name="c", subcore_axis_name="s")


@functools.partial(
    pl.kernel,
    out_type=jax.ShapeDtypeStruct((N, DP), jnp.float32),
    mesh=_mesh,
    scratch_types=[
        pltpu.VMEM((NB * CHUNK,), jnp.int32),
        pltpu.VMEM((NB, CHUNK, DP), jnp.float32),
        pltpu.VMEM_SHARED((V, DP), jnp.float32),
        pltpu.SemaphoreType.DMA,
        pltpu.SemaphoreType.DMA,
    ],
)
def _gather_kernel(idx_hbm, table_hbm, out_hbm, idx_v, rows_v, table_sh, gsem, wsem):
    cid = lax.axis_index("c")
    sid = lax.axis_index("s")
    wid = sid * NC + cid
    base = wid * B_PER_W

    @pl.when(sid == 0)
    def _stage():
        pltpu.sync_copy(table_hbm, table_sh)

    plsc.subcore_barrier()

    def body(g, carry):
        pltpu.sync_copy(idx_hbm.at[pl.ds(base + g * NB * CHUNK, NB * CHUNK)], idx_v)
        gathers = [
            pltpu.async_copy(
                table_sh.at[idx_v.at[pl.ds(b * CHUNK, CHUNK)]], rows_v.at[b], gsem
            )
            for b in range(NB)
        ]
        writes = []
        for b in range(NB):
            gathers[b].wait()
            off = base + (g * NB + b) * CHUNK
            writes.append(
                pltpu.async_copy(rows_v.at[b], out_hbm.at[pl.ds(off, CHUNK)], wsem)
            )
        for w in writes:
            w.wait()
        return carry

    lax.fori_loop(0, NOUTER, body, 0)


def kernel(pos_ids, table):
    idx = pos_ids.reshape(N).astype(jnp.int32)
    table_p = jnp.pad(table, ((0, 0), (0, DP - D)))
    out = _gather_kernel(idx, table_p)
    return out[:, :D].reshape(B, T, D)


# speedup vs baseline: 3.4146x; 1.0864x over previous
import functools

import jax
import jax.numpy as jnp
from jax import lax
from jax.experimental import pallas as pl
from jax.experimental.pallas import tpu as pltpu
from jax.experimental.pallas import tpu_sc as plsc

NC, NS = 2, 16
NW = NC * NS

B, T = 16384, 200
V, D = 1000, 50
DP = 64
N = B * T
B_PER_W = N // NW
CHUNK = 128
NB = 4
NCHUNK = B_PER_W // CHUNK
NOUTER = NCHUNK // NB

_mesh = plsc.VectorSubcoreMesh(core_axis_name="c", subcore_axis_name="s")


@functools.partial(
    pl.kernel,
    out_type=jax.ShapeDtypeStruct((N, DP), jnp.float32),
    mesh=_mesh,
    scratch_types=[
        pltpu.VMEM((NB * CHUNK,), jnp.int32),
        pltpu.VMEM((NB, CHUNK, DP), jnp.float32),
        pltpu.VMEM_SHARED((V, DP), jnp.float32),
        pltpu.SemaphoreType.DMA,
        pltpu.SemaphoreType.DMA,
    ],
)
def _gather_kernel(idx_hbm, table_hbm, out_hbm, idx_v, rows_v, table_sh, gsem, wsem):
    cid = lax.axis_index("c")
    sid = lax.axis_index("s")
    wid = sid * NC + cid
    base = wid * B_PER_W

    @pl.when(sid == 0)
    def _stage():
        pltpu.sync_copy(table_hbm, table_sh)

    plsc.subcore_barrier()

    def body(g, carry):
        pltpu.sync_copy(idx_hbm.at[pl.ds(base + g * NB * CHUNK, NB * CHUNK)], idx_v)
        gathers = [
            pltpu.async_copy(
                table_sh.at[idx_v.at[pl.ds(b * CHUNK, CHUNK)]], rows_v.at[b], gsem
            )
            for b in range(NB)
        ]
        writes = []
        for b in range(NB):
            gathers[b].wait()
            off = base + (g * NB + b) * CHUNK
            writes.append(
                pltpu.async_copy(rows_v.at[b], out_hbm.at[pl.ds(off, CHUNK)], wsem)
            )
        for w in writes:
            w.wait()
        return carry

    lax.fori_loop(0, NOUTER, body, 0)


def kernel(pos_ids, table):
    idx = pos_ids.reshape(N).astype(jnp.int32)
    table_p = jnp.pad(table, ((0, 0), (0, DP - D)))
    out = _gather_kernel(idx, table_p)
    return out[:, :D].reshape(B, T, D)
